# Initial kernel scaffold; baseline (speedup 1.0000x reference)
import jax
import jax.numpy as jnp
from jax.experimental import pallas as pl


def kernel(node_features, edge_features, targets, edge_index, W_dense, b_dense, W_edge, b_edge, W_out, b_out):
    raise NotImplementedError("write your pallas kernel here")


# speedup vs baseline: 1.9115x; 1.9115x over previous
import functools

import jax
import jax.numpy as jnp
from jax import lax
from jax.experimental import pallas as pl
from jax.experimental.pallas import tpu as pltpu
from jax.experimental.pallas import tpu_sc as plsc

N, E, D = 10000, 320000, 128

NC, NS, L = 2, 16, 16
NW = NC * NS
CH = 128
NCHUNK = E // CH
NPAD = 10240
ROWS_PER_SUB = NPAD // NS


def _mish(t):
    return t * jnp.tanh(jax.nn.softplus(t))


def _mm_mish_body(a_ref, w_ref, b_ref, o_ref):
    a = a_ref[...]
    w = w_ref[...]
    acc = lax.dot_general(a, w, (((1,), (1,)), ((), ())),
                          preferred_element_type=jnp.float32)
    o_ref[...] = _mish(acc + b_ref[...])


def _mm_mish(a, w, b, block_rows):
    rows = a.shape[0]
    grid = rows // block_rows
    return pl.pallas_call(
        _mm_mish_body,
        grid=(grid,),
        in_specs=[
            pl.BlockSpec((block_rows, D), lambda i: (i, 0)),
            pl.BlockSpec((D, D), lambda i: (0, 0)),
            pl.BlockSpec((1, D), lambda i: (0, 0)),
        ],
        out_specs=pl.BlockSpec((block_rows, D), lambda i: (i, 0)),
        out_shape=jax.ShapeDtypeStruct((rows, D), jnp.float32),
    )(a, w, b)


def _final_body(x_ref, a0_ref, a1_ref, w_ref, b_ref, o_ref):
    s = x_ref[...] + a0_ref[...] + a1_ref[...]
    acc = lax.dot_general(s, w_ref[...], (((1,), (1,)), ((), ())),
                          preferred_element_type=jnp.float32)
    o_ref[...] = _mish(acc + b_ref[...])


def _final(x, a0, a1, w, b, block_rows=1000):
    grid = N // block_rows
    return pl.pallas_call(
        _final_body,
        grid=(grid,),
        in_specs=[
            pl.BlockSpec((block_rows, D), lambda i: (i, 0)),
            pl.BlockSpec((block_rows, D), lambda i: (i, 0)),
            pl.BlockSpec((block_rows, D), lambda i: (i, 0)),
            pl.BlockSpec((D, D), lambda i: (0, 0)),
            pl.BlockSpec((1, D), lambda i: (0, 0)),
        ],
        out_specs=pl.BlockSpec((block_rows, D), lambda i: (i, 0)),
        out_shape=jax.ShapeDtypeStruct((N, D), jnp.float32),
    )(x, a0, a1, w, b)


def _sc_body(x_hbm, y_hbm, src_hbm, dst_hbm, out_hbm,
             srcv, dstv, gx, gy, acc, sem):
    cid = lax.axis_index("c")
    sid = lax.axis_index("s")
    wid = sid * NC + cid

    def _z(t, _):
        e = t >> 3
        j = (t & 7) * L
        gx[e, pl.ds(j, L)] = jnp.zeros((L,), jnp.float32)
        return 0
    lax.fori_loop(0, CH * D // L, _z, 0)
    base_rows = sid * ROWS_PER_SUB
    for kk in range(ROWS_PER_SUB // CH):
        pltpu.sync_copy(gx, acc.at[pl.ds(base_rows + kk * CH, CH)])
    plsc.subcore_barrier()

    def _chunk(i, _):
        k = wid + i * NW

        @pl.when(k < NCHUNK)
        def _():
            base = k * CH
            pltpu.sync_copy(src_hbm.at[pl.ds(base, CH)], srcv)
            pltpu.sync_copy(dst_hbm.at[pl.ds(base, CH)], dstv)
            gat = pltpu.async_copy(x_hbm.at[srcv], gx, sem)
            pltpu.sync_copy(y_hbm.at[pl.ds(base, CH)], gy)
            gat.wait()

            def _cmp(t, _):
                e = t >> 3
                j = (t & 7) * L
                a = gx[e, pl.ds(j, L)]
                b = gy[e, pl.ds(j, L)]
                gx[e, pl.ds(j, L)] = jnp.maximum(a + b, 0.0)
                return 0
            lax.fori_loop(0, CH * D // L, _cmp, 0)

            pltpu.sync_copy(gx, acc.at[dstv], add=True)
        return 0
    lax.fori_loop(0, (NCHUNK + NW - 1) // NW, _chunk, 0)
    plsc.subcore_barrier()

    pltpu.sync_copy(acc.at[pl.ds(base_rows, ROWS_PER_SUB)],
                    out_hbm.at[cid, pl.ds(base_rows, ROWS_PER_SUB)])


@functools.partial(jax.jit, static_argnums=())
def _sc_agg(x, y, src, dst):
    mesh = plsc.VectorSubcoreMesh(core_axis_name="c", subcore_axis_name="s")
    f = pl.kernel(
        _sc_body,
        out_type=jax.ShapeDtypeStruct((NC, NPAD, D), jnp.float32),
        mesh=mesh,
        scratch_types=[
            pltpu.VMEM((CH,), jnp.int32),
            pltpu.VMEM((CH,), jnp.int32),
            pltpu.VMEM((CH, D), jnp.float32),
            pltpu.VMEM((CH, D), jnp.float32),
            pltpu.VMEM_SHARED((NPAD, D), jnp.float32),
            pltpu.SemaphoreType.DMA,
        ],
    )
    return f(x, y, src, dst)


def kernel(node_features, edge_features, targets, edge_index,
           W_dense, b_dense, W_edge, b_edge, W_out, b_out):
    del targets
    x = _mm_mish(node_features, W_dense, b_dense.reshape(1, D), 1000)
    y = _mm_mish(edge_features, W_edge, b_edge.reshape(1, D), 2000)
    agg = _sc_agg(x, y, edge_index[0], edge_index[1])
    return _final(x, agg[0, :N], agg[1, :N], W_out, b_out.reshape(1, D))


# speedup vs baseline: 3.9327x; 2.0574x over previous
import jax
import jax.numpy as jnp
from jax import lax
from jax.experimental import pallas as pl
from jax.experimental.pallas import tpu as pltpu
from jax.experimental.pallas import tpu_sc as plsc

N, E, D = 10000, 320000, 128

NC, NS, L = 2, 16, 16
NW = NC * NS
CH = 80
NCHUNK = E // CH
PER_W = NCHUNK // NW
NPAIR = (PER_W + 2) // 2
NPAD = 10112
ROWS_PER_SUB = NPAD // NS


def _mish(t):
    return t * jnp.tanh(jax.nn.softplus(t))


def _mm_mish_body(a_ref, w_ref, b_ref, o_ref):
    a = a_ref[...]
    w = w_ref[...]
    acc = lax.dot_general(a, w, (((1,), (1,)), ((), ())),
                          preferred_element_type=jnp.float32)
    o_ref[...] = _mish(acc + b_ref[...])


def _mm_mish(a, w, b, block_rows):
    rows = a.shape[0]
    grid = rows // block_rows
    return pl.pallas_call(
        _mm_mish_body,
        grid=(grid,),
        in_specs=[
            pl.BlockSpec((block_rows, D), lambda i: (i, 0)),
            pl.BlockSpec((D, D), lambda i: (0, 0)),
            pl.BlockSpec((1, D), lambda i: (0, 0)),
        ],
        out_specs=pl.BlockSpec((block_rows, D), lambda i: (i, 0)),
        out_shape=jax.ShapeDtypeStruct((rows, D), jnp.float32),
    )(a, w, b)


def _final_body(x_ref, a0_ref, a1_ref, w_ref, b_ref, o_ref):
    s = x_ref[...] + a0_ref[...] + a1_ref[...]
    acc = lax.dot_general(s, w_ref[...], (((1,), (1,)), ((), ())),
                          preferred_element_type=jnp.float32)
    o_ref[...] = _mish(acc + b_ref[...])


def _final(x, a0, a1, w, b, block_rows=1000):
    grid = N // block_rows
    return pl.pallas_call(
        _final_body,
        grid=(grid,),
        in_specs=[
            pl.BlockSpec((block_rows, D), lambda i: (i, 0)),
            pl.BlockSpec((block_rows, D), lambda i: (i, 0)),
            pl.BlockSpec((block_rows, D), lambda i: (i, 0)),
            pl.BlockSpec((D, D), lambda i: (0, 0)),
            pl.BlockSpec((1, D), lambda i: (0, 0)),
        ],
        out_specs=pl.BlockSpec((block_rows, D), lambda i: (i, 0)),
        out_shape=jax.ShapeDtypeStruct((N, D), jnp.float32),
    )(x, a0, a1, w, b)


def _sc_body(x_hbm, y_hbm, src_hbm, dst_hbm, out_hbm,
             srcv0, srcv1, dstv0, dstv1, gx0, gx1, gy0, gy1, acc,
             sem_i0, sem_i1, sem_g0, sem_g1):
    cid = lax.axis_index("c")
    sid = lax.axis_index("s")
    wid = sid * NC + cid

    srcv = (srcv0, srcv1)
    dstv = (dstv0, dstv1)
    gx = (gx0, gx1)
    gy = (gy0, gy1)
    sem_i = (sem_i0, sem_i1)
    sem_g = (sem_g0, sem_g1)

    c0 = wid * PER_W

    def _z(t, _):
        e = t >> 3
        j = (t & 7) * L
        gx0[e, pl.ds(j, L)] = jnp.zeros((L,), jnp.float32)
        return 0
    lax.fori_loop(0, CH * D // L, _z, 0)
    base_rows = sid * ROWS_PER_SUB
    for kk in range(ROWS_PER_SUB // CH):
        pltpu.sync_copy(gx0, acc.at[pl.ds(base_rows + kk * CH, CH)])
    rem = ROWS_PER_SUB % CH
    if rem:
        pltpu.sync_copy(gx0.at[pl.ds(0, rem)],
                        acc.at[pl.ds(base_rows + (ROWS_PER_SUB // CH) * CH,
                                     rem)])
    plsc.subcore_barrier()

    def issue_idx(i, b):
        base = (c0 + i) * CH
        pltpu.async_copy(src_hbm.at[pl.ds(base, CH)], srcv[b], sem_i[b])
        pltpu.async_copy(dst_hbm.at[pl.ds(base, CH)], dstv[b], sem_i[b])

    def wait_idx(b):
        pltpu.make_async_copy(src_hbm.at[pl.ds(0, CH)], srcv[b],
                              sem_i[b]).wait()
        pltpu.make_async_copy(dst_hbm.at[pl.ds(0, CH)], dstv[b],
                              sem_i[b]).wait()

    def issue_gather(i, b):
        base = (c0 + i) * CH * D
        pltpu.async_copy(x_hbm.at[srcv[b]], gx[b], sem_g[b])
        pltpu.async_copy(y_hbm.at[pl.ds(base, CH * D)], gy[b], sem_g[b])

    def wait_gather(b):
        pltpu.make_async_copy(x_hbm.at[srcv[b]], gx[b], sem_g[b]).wait()
        pltpu.make_async_copy(y_hbm.at[pl.ds(0, CH * D)], gy[b],
                              sem_g[b]).wait()

    def compute(b):
        gxb = gx[b]
        gyb = gy[b]

        def _row(e, _):
            for t in range(D // L):
                u = gyb[pl.ds(e * D + t * L, L)]
                a = gxb[e, pl.ds(t * L, L)]
                gxb[e, pl.ds(t * L, L)] = jnp.maximum(a + u, 0.0)
            return 0
        lax.fori_loop(0, CH, _row, 0)

    issue_idx(0, 0)
    issue_idx(1, 1)
    wait_idx(0)
    issue_gather(0, 0)

    def _pair(t, _):
        for b in (0, 1):
            i = 2 * t + b

            @pl.when(i + 1 < PER_W)
            def _():
                wait_idx(1 - b)
                issue_gather(i + 1, 1 - b)

            @pl.when(i < PER_W)
            def _():
                wait_gather(b)
                compute(b)
                pltpu.sync_copy(gx[b], acc.at[dstv[b]], add=True)

            @pl.when(i + 2 < PER_W)
            def _():
                issue_idx(i + 2, b)
        return 0
    lax.fori_loop(0, NPAIR, _pair, 0)
    plsc.subcore_barrier()

    pltpu.sync_copy(acc.at[pl.ds(base_rows, ROWS_PER_SUB)],
                    out_hbm.at[cid, pl.ds(base_rows, ROWS_PER_SUB)])


def _sc_agg(x, y, src, dst):
    mesh = plsc.VectorSubcoreMesh(core_axis_name="c", subcore_axis_name="s")
    f = pl.kernel(
        _sc_body,
        out_type=jax.ShapeDtypeStruct((NC, NPAD, D), jnp.float32),
        mesh=mesh,
        scratch_types=[
            pltpu.VMEM((CH,), jnp.int32),
            pltpu.VMEM((CH,), jnp.int32),
            pltpu.VMEM((CH,), jnp.int32),
            pltpu.VMEM((CH,), jnp.int32),
            pltpu.VMEM((CH, D), jnp.float32),
            pltpu.VMEM((CH, D), jnp.float32),
            pltpu.VMEM((CH * D,), jnp.float32),
            pltpu.VMEM((CH * D,), jnp.float32),
            pltpu.VMEM_SHARED((NPAD, D), jnp.float32),
            pltpu.SemaphoreType.DMA,
            pltpu.SemaphoreType.DMA,
            pltpu.SemaphoreType.DMA,
            pltpu.SemaphoreType.DMA,
        ],
    )
    return f(x, y, src, dst)


def kernel(node_features, edge_features, targets, edge_index,
           W_dense, b_dense, W_edge, b_edge, W_out, b_out):
    del targets
    x = _mm_mish(node_features, W_dense, b_dense.reshape(1, D), 1000)
    y = _mm_mish(edge_features, W_edge, b_edge.reshape(1, D), 2000)
    agg = _sc_agg(x, y.reshape(E * D), edge_index[0], edge_index[1])
    return _final(x, agg[0, :N], agg[1, :N], W_out, b_out.reshape(1, D))


# speedup vs baseline: 4.1409x; 1.0530x over previous
import jax
import jax.numpy as jnp
from jax import lax
from jax.experimental import pallas as pl
from jax.experimental.pallas import tpu as pltpu
from jax.experimental.pallas import tpu_sc as plsc

N, E, D = 10000, 320000, 128

NC, NS, L = 2, 16, 16
NW = NC * NS
CH = 80
NCHUNK = E // CH
PER_W = NCHUNK // NW
NPAIR = (PER_W + 2) // 2
NPAD = 10112
ROWS_PER_SUB = NPAD // NS


def _mish(t):
    u = 1.0 + jnp.exp(t)
    return t * (1.0 - 2.0 / (u * u + 1.0))


def _mm_mish_body(a_ref, w_ref, b_ref, o_ref):
    a = a_ref[...]
    w = w_ref[...]
    acc = lax.dot_general(a, w, (((1,), (1,)), ((), ())),
                          preferred_element_type=jnp.float32)
    o_ref[...] = _mish(acc + b_ref[...])


def _mm_mish(a, w, b, block_rows):
    rows = a.shape[0]
    grid = rows // block_rows
    return pl.pallas_call(
        _mm_mish_body,
        grid=(grid,),
        in_specs=[
            pl.BlockSpec((block_rows, D), lambda i: (i, 0)),
            pl.BlockSpec((D, D), lambda i: (0, 0)),
            pl.BlockSpec((1, D), lambda i: (0, 0)),
        ],
        out_specs=pl.BlockSpec((block_rows, D), lambda i: (i, 0)),
        out_shape=jax.ShapeDtypeStruct((rows, D), jnp.float32),
    )(a, w, b)


def _final_body(x_ref, a0_ref, a1_ref, w_ref, b_ref, o_ref):
    s = x_ref[...] + a0_ref[...] + a1_ref[...]
    acc = lax.dot_general(s, w_ref[...], (((1,), (1,)), ((), ())),
                          preferred_element_type=jnp.float32)
    o_ref[...] = _mish(acc + b_ref[...])


def _final(x, a0, a1, w, b, block_rows=1000):
    grid = N // block_rows
    return pl.pallas_call(
        _final_body,
        grid=(grid,),
        in_specs=[
            pl.BlockSpec((block_rows, D), lambda i: (i, 0)),
            pl.BlockSpec((block_rows, D), lambda i: (i, 0)),
            pl.BlockSpec((block_rows, D), lambda i: (i, 0)),
            pl.BlockSpec((D, D), lambda i: (0, 0)),
            pl.BlockSpec((1, D), lambda i: (0, 0)),
        ],
        out_specs=pl.BlockSpec((block_rows, D), lambda i: (i, 0)),
        out_shape=jax.ShapeDtypeStruct((N, D), jnp.float32),
    )(x, a0, a1, w, b)


def _sc_body(x_hbm, y_hbm, src_hbm, dst_hbm, out_hbm,
             srcv0, srcv1, dstv0, dstv1, gx0, gx1, gy0, gy1, acc,
             sem_i0, sem_i1, sem_g0, sem_g1):
    cid = lax.axis_index("c")
    sid = lax.axis_index("s")
    wid = sid * NC + cid

    srcv = (srcv0, srcv1)
    dstv = (dstv0, dstv1)
    gx = (gx0, gx1)
    gy = (gy0, gy1)
    sem_i = (sem_i0, sem_i1)
    sem_g = (sem_g0, sem_g1)

    c0 = wid * PER_W

    def _z(t, _):
        e = t >> 3
        j = (t & 7) * L
        gx0[e, pl.ds(j, L)] = jnp.zeros((L,), jnp.float32)
        return 0
    lax.fori_loop(0, CH * D // L, _z, 0)
    base_rows = sid * ROWS_PER_SUB
    for kk in range(ROWS_PER_SUB // CH):
        pltpu.sync_copy(gx0, acc.at[pl.ds(base_rows + kk * CH, CH)])
    rem = ROWS_PER_SUB % CH
    if rem:
        pltpu.sync_copy(gx0.at[pl.ds(0, rem)],
                        acc.at[pl.ds(base_rows + (ROWS_PER_SUB // CH) * CH,
                                     rem)])
    plsc.subcore_barrier()

    def issue_idx(i, b):
        base = (c0 + i) * CH
        pltpu.async_copy(src_hbm.at[pl.ds(base, CH)], srcv[b], sem_i[b])
        pltpu.async_copy(dst_hbm.at[pl.ds(base, CH)], dstv[b], sem_i[b])

    def wait_idx(b):
        pltpu.make_async_copy(src_hbm.at[pl.ds(0, CH)], srcv[b],
                              sem_i[b]).wait()
        pltpu.make_async_copy(dst_hbm.at[pl.ds(0, CH)], dstv[b],
                              sem_i[b]).wait()

    def issue_gather(i, b):
        base = (c0 + i) * CH * D
        pltpu.async_copy(x_hbm.at[srcv[b]], gx[b], sem_g[b])
        pltpu.async_copy(y_hbm.at[pl.ds(base, CH * D)], gy[b], sem_g[b])

    def wait_gather(b):
        pltpu.make_async_copy(x_hbm.at[srcv[b]], gx[b], sem_g[b]).wait()
        pltpu.make_async_copy(y_hbm.at[pl.ds(0, CH * D)], gy[b],
                              sem_g[b]).wait()

    def compute(b):
        gxb = gx[b]
        gyb = gy[b]

        def _row(e, _):
            for t in range(D // L):
                u = gyb[pl.ds(e * D + t * L, L)]
                a = gxb[e, pl.ds(t * L, L)]
                gxb[e, pl.ds(t * L, L)] = jnp.maximum(a + u, 0.0)
            return 0
        lax.fori_loop(0, CH, _row, 0)

    issue_idx(0, 0)
    issue_idx(1, 1)
    wait_idx(0)
    issue_gather(0, 0)

    def _pair(t, _):
        for b in (0, 1):
            i = 2 * t + b

            @pl.when(i + 1 < PER_W)
            def _():
                wait_idx(1 - b)
                issue_gather(i + 1, 1 - b)

            @pl.when(i < PER_W)
            def _():
                wait_gather(b)
                compute(b)
                pltpu.sync_copy(gx[b], acc.at[dstv[b]], add=True)

            @pl.when(i + 2 < PER_W)
            def _():
                issue_idx(i + 2, b)
        return 0
    lax.fori_loop(0, NPAIR, _pair, 0)
    plsc.subcore_barrier()

    pltpu.sync_copy(acc.at[pl.ds(base_rows, ROWS_PER_SUB)],
                    out_hbm.at[cid, pl.ds(base_rows, ROWS_PER_SUB)])


def _sc_agg(x, y, src, dst):
    mesh = plsc.VectorSubcoreMesh(core_axis_name="c", subcore_axis_name="s")
    f = pl.kernel(
        _sc_body,
        out_type=jax.ShapeDtypeStruct((NC, NPAD, D), jnp.float32),
        mesh=mesh,
        scratch_types=[
            pltpu.VMEM((CH,), jnp.int32),
            pltpu.VMEM((CH,), jnp.int32),
            pltpu.VMEM((CH,), jnp.int32),
            pltpu.VMEM((CH,), jnp.int32),
            pltpu.VMEM((CH, D), jnp.float32),
            pltpu.VMEM((CH, D), jnp.float32),
            pltpu.VMEM((CH * D,), jnp.float32),
            pltpu.VMEM((CH * D,), jnp.float32),
            pltpu.VMEM_SHARED((NPAD, D), jnp.float32),
            pltpu.SemaphoreType.DMA,
            pltpu.SemaphoreType.DMA,
            pltpu.SemaphoreType.DMA,
            pltpu.SemaphoreType.DMA,
        ],
    )
    return f(x, y, src, dst)


def kernel(node_features, edge_features, targets, edge_index,
           W_dense, b_dense, W_edge, b_edge, W_out, b_out):
    del targets
    x = _mm_mish(node_features, W_dense, b_dense.reshape(1, D), 1000)
    y = _mm_mish(edge_features, W_edge, b_edge.reshape(1, D), 2000)
    agg = _sc_agg(x, y.reshape(E * D), edge_index[0], edge_index[1])
    return _final(x, agg[0, :N], agg[1, :N], W_out, b_out.reshape(1, D))
